# baseline (speedup 1.0000x reference)
import functools

import jax
import jax.numpy as jnp
from jax import lax
from jax.experimental import pallas as pl
from jax.experimental.pallas import tpu as pltpu
from jax.experimental.pallas import tpu_sc as plsc

VOCAB = 1000
EMBED_DIM = 64
BATCH = 4096
HIST = 50

_NC = 2
_NS = 16
_NW = _NC * _NS

_BC = 128
_NCHUNK = BATCH // _BC
_DG = 8
_HN = 13
_NBUF = 4


@functools.partial(
    pl.kernel,
    mesh=plsc.VectorSubcoreMesh(core_axis_name="c", subcore_axis_name="s"),
    out_type=jax.ShapeDtypeStruct((HIST, _DG, _NCHUNK, 8, _BC), jnp.float32),
    scratch_types=[
    ] + [pltpu.VMEM((VOCAB,), jnp.float32)] * _DG + [
        pltpu.VMEM((_NCHUNK, _HN, _BC), jnp.int32),
        pltpu.VMEM((_NBUF, _HN, _DG, _BC), jnp.float32),
        pltpu.SemaphoreType.DMA,
        pltpu.SemaphoreType.DMA,
    ] + [pltpu.SemaphoreType.DMA] * _NBUF,
    compiler_params=pltpu.CompilerParams(
        use_tc_tiling_on_sc=False, needs_layout_passes=False
    ),
)
def _tgather_kernel(xtr_hbm, tflat_hbm, out_hbm, t0, t1, t2, t3, t4, t5,
                    t6, t7, idx_v, buf_v, isem, isem2, *wsems):
    tabs = (t0, t1, t2, t3, t4, t5, t6, t7)
    wid = lax.axis_index("s") * _NC + lax.axis_index("c")
    dg = lax.rem(wid, _DG)
    hq = wid // _DG
    d0 = pl.multiple_of(dg * 8, 8)
    h0 = hq * _HN - 2 * (hq // 3)

    for dd in range(_DG):
        pltpu.async_copy(
            tflat_hbm.at[pl.ds((d0 + dd) * VOCAB, VOCAB)], tabs[dd], isem
        )
    pltpu.async_copy(
        xtr_hbm.at[pl.ds(0, _NBUF), pl.ds(h0, _HN)],
        idx_v.at[pl.ds(0, _NBUF)],
        isem,
    )
    pltpu.async_copy(
        xtr_hbm.at[pl.ds(_NBUF, _NCHUNK - _NBUF), pl.ds(h0, _HN)],
        idx_v.at[pl.ds(_NBUF, _NCHUNK - _NBUF)],
        isem2,
    )
    for dd in range(_DG):
        pltpu.make_async_copy(
            tflat_hbm.at[pl.ds(0, VOCAB)], tabs[dd], isem
        ).wait()
    pltpu.make_async_copy(
        xtr_hbm.at[pl.ds(0, _NBUF), pl.ds(0, _HN)],
        idx_v.at[pl.ds(0, _NBUF)],
        isem,
    ).wait()

    def compute_chunk(j, p):
        @plsc.parallel_loop(0, _HN, step=1)
        def hh_body(hh):
            for bl in range(_BC // 16):
                iv = idx_v[j, hh, pl.ds(bl * 16, 16)]
                for dd in range(_DG):
                    buf_v[p, hh, dd, pl.ds(bl * 16, 16)] = plsc.load_gather(
                        tabs[dd], [iv]
                    )

    def body(g, carry):
        @pl.when(g == 1)
        def _():
            pltpu.make_async_copy(
                xtr_hbm.at[pl.ds(_NBUF, _NCHUNK - _NBUF), pl.ds(0, _HN)],
                idx_v.at[pl.ds(_NBUF, _NCHUNK - _NBUF)],
                isem2,
            ).wait()

        for p in range(_NBUF):
            j = _NBUF * g + p
            @pl.when(g > 0)
            def _():
                pltpu.make_async_copy(
                    buf_v.at[p],
                    out_hbm.at[pl.ds(0, _HN), 0, 0],
                    wsems[p],
                ).wait()

            compute_chunk(j, p)

            pltpu.async_copy(
                buf_v.at[p],
                out_hbm.at[pl.ds(h0, _HN), dg, j],
                wsems[p],
            )

        return carry

    lax.fori_loop(0, _NCHUNK // _NBUF, body, 0)

    for p in range(_NBUF):
        pltpu.make_async_copy(
            buf_v.at[p],
            out_hbm.at[pl.ds(0, _HN), 0, 0],
            wsems[p],
        ).wait()


def kernel(x, table):
    tflat = table.T.reshape(-1)
    xtr = jnp.pad(
        x.T.reshape(HIST, _NCHUNK, _BC).transpose(1, 0, 2),
        ((0, 0), (0, 6), (0, 0)),
    )
    t5 = _tgather_kernel(xtr, tflat)
    return t5.transpose(2, 4, 0, 1, 3).reshape(BATCH, HIST, EMBED_DIM)


# speedup vs baseline: 1.4735x; 1.4735x over previous
import functools

import jax
import jax.numpy as jnp
from jax import lax
from jax.experimental import pallas as pl
from jax.experimental.pallas import tpu as pltpu
from jax.experimental.pallas import tpu_sc as plsc

VOCAB = 1000
EMBED_DIM = 64
BATCH = 4096
HIST = 50

_NC = 2
_NS = 16
_NW = _NC * _NS

_BC = 128
_NCHUNK = BATCH // _BC
_DG = 8
_HN = 13
_NBUF = 4


@functools.partial(
    pl.kernel,
    mesh=plsc.VectorSubcoreMesh(core_axis_name="c", subcore_axis_name="s"),
    out_type=jax.ShapeDtypeStruct((HIST, _DG, _NCHUNK, 8, _BC), jnp.float32),
    scratch_types=[
    ] + [pltpu.VMEM((VOCAB,), jnp.float32)] * _DG + [
        pltpu.VMEM((_NCHUNK, _HN, _BC), jnp.int32),
        pltpu.VMEM((_NBUF, _HN, _DG, _BC), jnp.float32),
        pltpu.SemaphoreType.DMA,
        pltpu.SemaphoreType.DMA,
    ] + [pltpu.SemaphoreType.DMA] * _NBUF,
    compiler_params=pltpu.CompilerParams(
        use_tc_tiling_on_sc=False, needs_layout_passes=False
    ),
)
def _tgather_kernel(xtr_hbm, tflat_hbm, out_hbm, t0, t1, t2, t3, t4, t5,
                    t6, t7, idx_v, buf_v, isem, isem2, *wsems):
    tabs = (t0, t1, t2, t3, t4, t5, t6, t7)
    wid = lax.axis_index("s") * _NC + lax.axis_index("c")
    dg = lax.rem(wid, _DG)
    hq = wid // _DG
    d0 = pl.multiple_of(dg * 8, 8)
    h0 = hq * _HN - 2 * (hq // 3)

    for dd in range(_DG):
        pltpu.async_copy(
            tflat_hbm.at[pl.ds((d0 + dd) * VOCAB, VOCAB)], tabs[dd], isem
        )
    pltpu.async_copy(
        xtr_hbm.at[pl.ds(0, _NBUF), pl.ds(h0, _HN)],
        idx_v.at[pl.ds(0, _NBUF)],
        isem,
    )
    pltpu.async_copy(
        xtr_hbm.at[pl.ds(_NBUF, _NCHUNK - _NBUF), pl.ds(h0, _HN)],
        idx_v.at[pl.ds(_NBUF, _NCHUNK - _NBUF)],
        isem2,
    )
    for dd in range(_DG):
        pltpu.make_async_copy(
            tflat_hbm.at[pl.ds(0, VOCAB)], tabs[dd], isem
        ).wait()
    pltpu.make_async_copy(
        xtr_hbm.at[pl.ds(0, _NBUF), pl.ds(0, _HN)],
        idx_v.at[pl.ds(0, _NBUF)],
        isem,
    ).wait()

    def compute_chunk(j, p):
        @plsc.parallel_loop(0, _HN * (_BC // 16), step=1)
        def hb_body(t):
            hh = t // (_BC // 16)
            bl = t % (_BC // 16)
            iv = idx_v[j, hh, pl.ds(bl * 16, 16)]
            for dd in range(_DG):
                buf_v[p, hh, dd, pl.ds(bl * 16, 16)] = plsc.load_gather(
                    tabs[dd], [iv]
                )

    def body(g, carry):
        @pl.when(g == 1)
        def _():
            pltpu.make_async_copy(
                xtr_hbm.at[pl.ds(_NBUF, _NCHUNK - _NBUF), pl.ds(0, _HN)],
                idx_v.at[pl.ds(_NBUF, _NCHUNK - _NBUF)],
                isem2,
            ).wait()

        for p in range(_NBUF):
            j = _NBUF * g + p
            @pl.when(g > 0)
            def _():
                pltpu.make_async_copy(
                    buf_v.at[p],
                    out_hbm.at[pl.ds(0, _HN), 0, 0],
                    wsems[p],
                ).wait()

            compute_chunk(j, p)

            pltpu.async_copy(
                buf_v.at[p],
                out_hbm.at[pl.ds(h0, _HN), dg, j],
                wsems[p],
            )

        return carry

    lax.fori_loop(0, _NCHUNK // _NBUF, body, 0)

    for p in range(_NBUF):
        pltpu.make_async_copy(
            buf_v.at[p],
            out_hbm.at[pl.ds(0, _HN), 0, 0],
            wsems[p],
        ).wait()


def kernel(x, table):
    tflat = table.T.reshape(-1)
    xtr = jnp.pad(
        x.T.reshape(HIST, _NCHUNK, _BC).transpose(1, 0, 2),
        ((0, 0), (0, 6), (0, 0)),
    )
    t5 = _tgather_kernel(xtr, tflat)
    return t5.transpose(2, 4, 0, 1, 3).reshape(BATCH, HIST, EMBED_DIM)
